# baseline (speedup 1.0000x reference)
import jax
import jax.numpy as jnp
from jax import lax
from jax.experimental import pallas as pl
from jax.experimental.pallas import tpu as pltpu
from jax.experimental.pallas import tpu_sc as plsc

N_USER = 20000
N_ITEM = 30000
N = N_USER + N_ITEM
E = 800000
D = 64
L = 3
H = D // 2

NUM_TILES = 16
CHUNK = 256
SUB = 128
SUPER = 2048
SUPERS = 25
EDGES_PER_TILE = SUPER * SUPERS
E_PAD = EDGES_PER_TILE * NUM_TILES
N_PAD = 50176
NODES_PER_TILE = N_PAD // NUM_TILES
WB = 224


def _spmm_body(table2, gidx2, dst2, w, out,
               gidx_sv, w_sv, dst_sv, rows0, rows1, acc, sg0, sg1, ss0, ss1):
    h = jnp.int32(lax.axis_index("c"))
    s = jnp.int32(lax.axis_index("s"))
    rows = (rows0, rows1)
    sg = (sg0, sg1)
    ss = (ss0, ss1)
    zero16 = jnp.zeros((16,), jnp.float32)

    @pl.loop(0, WB, unroll=8)
    def _zero(i):
        rows0[i, pl.ds(0, 16)] = zero16
        rows0[i, pl.ds(16, 16)] = zero16

    r0 = s * NODES_PER_TILE
    for k in range(NODES_PER_TILE // WB):
        pltpu.sync_copy(rows0.at[pl.ds(0, WB)], acc.at[pl.ds(r0 + k * WB, WB)])
    plsc.subcore_barrier()

    e0 = s * EDGES_PER_TILE
    n_chunks = SUPER // CHUNK

    def issue_gather(k):
        b = k % 2
        for jj in range(CHUNK // SUB):
            pltpu.async_copy(
                table2.at[gidx_sv.at[pl.ds(k * CHUNK + jj * SUB, SUB)]],
                rows[b].at[pl.ds(jj * SUB, SUB)],
                sg[b],
            )

    def wait_gather(k):
        b = k % 2
        pltpu.make_async_copy(table2.at[pl.ds(0, CHUNK)], rows[b], sg[b]).wait()

    def issue_scatter(k):
        b = k % 2
        for jj in range(CHUNK // SUB):
            pltpu.async_copy(
                rows[b].at[pl.ds(jj * SUB, SUB)],
                acc.at[dst_sv.at[k * (CHUNK // SUB) + jj]],
                ss[b],
                add=True,
            )

    def wait_scatter(k):
        b = k % 2
        pltpu.make_async_copy(table2.at[pl.ds(0, CHUNK)], rows[b], ss[b]).wait()

    @pl.loop(0, SUPERS)
    def _super(si):
        eb = e0 + si * SUPER
        ebsub = s * (EDGES_PER_TILE // SUB) + si * (SUPER // SUB)
        pltpu.sync_copy(gidx2.at[h, pl.ds(eb, SUPER)], gidx_sv)
        pltpu.sync_copy(w.at[pl.ds(eb, SUPER)], w_sv)
        pltpu.sync_copy(dst2.at[pl.ds(ebsub, SUPER // SUB)], dst_sv)

        issue_gather(0)
        for k in range(n_chunks):
            if k < n_chunks - 1:
                if k >= 1:
                    wait_scatter(k - 1)
                issue_gather(k + 1)
            wait_gather(k)
            b = k % 2

            @pl.loop(0, CHUNK // 16)
            def _scale(g):
                wvec = w_sv[pl.ds(k * CHUNK + g * 16, 16)]
                for l in range(16):
                    e = g * 16 + l
                    wv = jnp.broadcast_to(wvec[l], (16,))
                    rows[b][e, pl.ds(0, 16)] = rows[b][e, pl.ds(0, 16)] * wv
                    rows[b][e, pl.ds(16, 16)] = rows[b][e, pl.ds(16, 16)] * wv

            issue_scatter(k)
        wait_scatter(n_chunks - 2)
        wait_scatter(n_chunks - 1)

    plsc.subcore_barrier()

    for k in range(NODES_PER_TILE // WB):
        pltpu.sync_copy(acc.at[pl.ds(r0 + k * WB, WB)], rows0.at[pl.ds(0, WB)])
        pltpu.sync_copy(rows0.at[pl.ds(0, WB)], out.at[h, pl.ds(r0 + k * WB, WB)])


_spmm = pl.kernel(
    _spmm_body,
    out_type=jax.ShapeDtypeStruct((2, N_PAD, H), jnp.float32),
    mesh=plsc.VectorSubcoreMesh(core_axis_name="c", subcore_axis_name="s"),
    scratch_types=[
        pltpu.VMEM((SUPER,), jnp.int32),
        pltpu.VMEM((SUPER,), jnp.float32),
        pltpu.VMEM((SUPER // SUB, SUB), jnp.int32),
        pltpu.VMEM((CHUNK, H), jnp.float32),
        pltpu.VMEM((CHUNK, H), jnp.float32),
        pltpu.VMEM_SHARED((N_PAD, H), jnp.float32),
        pltpu.SemaphoreType.DMA,
        pltpu.SemaphoreType.DMA,
        pltpu.SemaphoreType.DMA,
        pltpu.SemaphoreType.DMA,
    ],
    compiler_params=pltpu.CompilerParams(use_tc_tiling_on_sc=False),
)


BN = 1000


def _dense_body(a0, a1, el, wf, bf, wb, bb, enew_ref, norm_ref):
    agg = jnp.concatenate([a0[0], a1[0]], axis=1)
    el_v = el[...]
    front = agg + el_v
    fc = front @ wf[...] + bf[...]
    fc = jnp.where(fc >= 0, fc, 0.01 * fc)
    back = (el_v * front) @ wb[...] + bb[...]
    back = jnp.where(back >= 0, back, 0.01 * back)
    enew = fc + back
    nrm = jnp.sqrt(jnp.sum(enew * enew, axis=1, keepdims=True))
    norm_ref[...] = enew / jnp.maximum(nrm, 1e-12)
    enew_ref[...] = enew


_dense = pl.pallas_call(
    _dense_body,
    grid=(N // BN,),
    in_specs=[
        pl.BlockSpec((1, BN, H), lambda i: (0, i, 0)),
        pl.BlockSpec((1, BN, H), lambda i: (1, i, 0)),
        pl.BlockSpec((BN, D), lambda i: (i, 0)),
        pl.BlockSpec((D, D), lambda i: (0, 0)),
        pl.BlockSpec((1, D), lambda i: (0, 0)),
        pl.BlockSpec((D, D), lambda i: (0, 0)),
        pl.BlockSpec((1, D), lambda i: (0, 0)),
    ],
    out_specs=[
        pl.BlockSpec((BN, D), lambda i: (i, 0)),
        pl.BlockSpec((BN, D), lambda i: (i, 0)),
    ],
    out_shape=[
        jax.ShapeDtypeStruct((N, D), jnp.float32),
        jax.ShapeDtypeStruct((N, D), jnp.float32),
    ],
    compiler_params=pltpu.CompilerParams(
        dimension_semantics=("arbitrary",),
    ),
)


def kernel(H_edge_index, H_edge_weight, user_emb, item_emb, Wf, bf, Wb, bb):
    E_l = jnp.concatenate([user_emb, item_emb], axis=0)
    src = H_edge_index[0].astype(jnp.int32)
    dst = H_edge_index[1].astype(jnp.int32)
    w = H_edge_weight.astype(jnp.float32)

    pad = E_PAD - E
    src_p = jnp.pad(src, (0, pad))
    gidx2 = jnp.stack([src_p * 2, src_p * 2 + 1])
    dst_p = jnp.pad(dst, (0, pad)).reshape(E_PAD // SUB, SUB)
    w_p = jnp.pad(w, (0, pad))

    outs = [E_l]
    for i in range(L):
        table2 = E_l.reshape(2 * N, H)
        agg2 = _spmm(table2, gidx2, dst_p, w_p)[:, :N, :]
        E_l, nrm = _dense(
            agg2, agg2, E_l,
            Wf[i], bf[i].reshape(1, D), Wb[i], bb[i].reshape(1, D),
        )
        outs.append(nrm)

    all_emb = jnp.concatenate(outs, axis=1)
    return all_emb[:N_USER], all_emb[N_USER:]
